# baseline (speedup 1.0000x reference)
import functools

import jax
import jax.numpy as jnp
from jax.experimental import pallas as pl
from jax.experimental.pallas import tpu as pltpu

_C = 512
_NB = 8


def _body(rows, seq_len, dim, in_hbm, emb_hbm, out_hbm,
          emb_v, in_bufs, out_bufs, esem, isems, osems):
    nch = rows // _C
    nehalf = seq_len // _C
    edescs = [
        pltpu.make_async_copy(emb_hbm.at[pl.ds(h * _C, _C)],
                              emb_v.at[pl.ds(h * _C, _C)], esem)
        for h in range(nehalf)
    ]
    in_descs = [None] * nch
    out_descs = [None] * nch
    for c in range(min(_NB, nch)):
        in_descs[c] = pltpu.make_async_copy(
            in_hbm.at[pl.ds(c * _C, _C)], in_bufs[c % _NB], isems[c % _NB])
    for c in range(min(_NB, nch)):
        if c < nehalf:
            edescs[c].start()
        in_descs[c].start()
    for h in range(min(_NB, nch), nehalf):
        edescs[h].start()
    for c in range(nch):
        s = c % _NB
        if c < nehalf:
            edescs[c].wait()
        in_descs[c].wait()
        if c >= _NB:
            out_descs[c - _NB].wait()
        out_bufs[s][...] = in_bufs[s][...] + emb_v[pl.ds((c * _C) % seq_len, _C), :]
        out_descs[c] = pltpu.make_async_copy(
            out_bufs[s], out_hbm.at[pl.ds(c * _C, _C)], osems[s])
        out_descs[c].start()
        if c + _NB < nch:
            in_descs[c + _NB] = pltpu.make_async_copy(
                in_hbm.at[pl.ds((c + _NB) * _C, _C)], in_bufs[s], isems[s])
            in_descs[c + _NB].start()
    for c in range(max(0, nch - _NB), nch):
        out_descs[c].wait()


def kernel(inputs, embeddings):
    batch, seq_len, dim = inputs.shape
    rows = batch * seq_len
    in_flat = inputs.reshape(rows, dim)
    pos = embeddings[:seq_len]
    out = pl.pallas_call(
        functools.partial(_body, rows, seq_len, dim),
        in_specs=[
            pl.BlockSpec(memory_space=pl.ANY),
            pl.BlockSpec(memory_space=pl.ANY),
        ],
        out_specs=pl.BlockSpec(memory_space=pl.ANY),
        out_shape=jax.ShapeDtypeStruct((rows, dim), inputs.dtype),
        scratch_shapes=[
            pltpu.VMEM((seq_len, dim), jnp.float32),
            [pltpu.VMEM((_C, dim), jnp.float32) for _ in range(_NB)],
            [pltpu.VMEM((_C, dim), jnp.float32) for _ in range(_NB)],
            pltpu.SemaphoreType.DMA,
            [pltpu.SemaphoreType.DMA for _ in range(_NB)],
            [pltpu.SemaphoreType.DMA for _ in range(_NB)],
        ],
        compiler_params=pltpu.CompilerParams(
            vmem_limit_bytes=100 * 1024 * 1024,
        ),
    )(in_flat, pos)
    return out.reshape(batch, seq_len, dim)


# speedup vs baseline: 1.0576x; 1.0576x over previous
import functools

import jax
import jax.numpy as jnp
from jax.experimental import pallas as pl
from jax.experimental.pallas import tpu as pltpu

_C = 512
_NB = 8


def _body(rows, seq_len, dim, in_hbm, emb_hbm, out_hbm,
          emb_v, in_bufs, out_bufs, esem, isems, osems):
    nch = rows // _C
    nehalf = seq_len // _C
    edescs = [
        pltpu.make_async_copy(emb_hbm.at[pl.ds(h * _C, _C)],
                              emb_v.at[pl.ds(h * _C, _C)], esem)
        for h in range(nehalf)
    ]
    in_descs = [None] * nch
    out_descs = [None] * nch
    for c in range(min(_NB, nch)):
        in_descs[c] = pltpu.make_async_copy(
            in_hbm.at[pl.ds(c * _C, _C)], in_bufs[c % _NB], isems[c % _NB])
        in_descs[c].start()
    for d in edescs:
        d.start()
    for c in range(nch):
        s = c % _NB
        if c < nehalf:
            edescs[c].wait()
        in_descs[c].wait()
        if c >= _NB:
            out_descs[c - _NB].wait()
        out_bufs[s][...] = in_bufs[s][...] + emb_v[pl.ds((c * _C) % seq_len, _C), :]
        out_descs[c] = pltpu.make_async_copy(
            out_bufs[s], out_hbm.at[pl.ds(c * _C, _C)], osems[s])
        out_descs[c].start()
        if c + _NB < nch:
            in_descs[c + _NB] = pltpu.make_async_copy(
                in_hbm.at[pl.ds((c + _NB) * _C, _C)], in_bufs[s], isems[s])
            in_descs[c + _NB].start()
    for c in range(max(0, nch - _NB), nch):
        out_descs[c].wait()


def kernel(inputs, embeddings):
    batch, seq_len, dim = inputs.shape
    rows = batch * seq_len
    in_flat = inputs.reshape(rows, dim)
    pos = embeddings[:seq_len]
    out = pl.pallas_call(
        functools.partial(_body, rows, seq_len, dim),
        in_specs=[
            pl.BlockSpec(memory_space=pl.ANY),
            pl.BlockSpec(memory_space=pl.ANY),
        ],
        out_specs=pl.BlockSpec(memory_space=pl.ANY),
        out_shape=jax.ShapeDtypeStruct((rows, dim), inputs.dtype),
        scratch_shapes=[
            pltpu.VMEM((seq_len, dim), jnp.float32),
            [pltpu.VMEM((_C, dim), jnp.float32) for _ in range(_NB)],
            [pltpu.VMEM((_C, dim), jnp.float32) for _ in range(_NB)],
            pltpu.SemaphoreType.DMA,
            [pltpu.SemaphoreType.DMA for _ in range(_NB)],
            [pltpu.SemaphoreType.DMA for _ in range(_NB)],
        ],
        compiler_params=pltpu.CompilerParams(
            vmem_limit_bytes=100 * 1024 * 1024,
        ),
    )(in_flat, pos)
    return out.reshape(batch, seq_len, dim)
